# baseline (speedup 1.0000x reference)
import functools

import jax
import jax.numpy as jnp
from jax import lax
from jax.experimental import pallas as pl
from jax.experimental.pallas import tpu as pltpu
from jax.experimental.pallas import tpu_sc as plsc

_GW = 128


def _sc_gather(sec_flat, fst16, idx):
    n = idx.shape[1]
    e = sec_flat.shape[1]
    mesh = plsc.VectorSubcoreMesh(core_axis_name="c", subcore_axis_name="s")

    @functools.partial(
        pl.kernel,
        out_type=[
            jax.ShapeDtypeStruct((n, e), jnp.float32),
            jax.ShapeDtypeStruct((n,), jnp.float32),
        ],
        mesh=mesh,
        scratch_types=[
            pltpu.VMEM((_GW,), jnp.int32),
            pltpu.VMEM((_GW, 16), jnp.float32),
        ],
        compiler_params=pltpu.CompilerParams(use_tc_tiling_on_sc=False,
                                             needs_layout_passes=False),
    )
    def k(sec_hbm, fst_hbm, i_hbm, osec_hbm, ofst_hbm, ridx_v, rows_v):
        def body(i_vmem, osec_vmem, ofst_vmem):
            pltpu.sync_copy(sec_hbm.at[i_vmem.at[0]], osec_vmem)

            @pl.loop(0, _GW, step=16)
            def _(c):
                ridx_v[pl.ds(c, 16)] = jax.lax.shift_right_logical(
                    i_vmem[0, pl.ds(c, 16)], 4)

            pltpu.sync_copy(fst_hbm.at[ridx_v], rows_v)
            lane16 = jax.lax.iota(jnp.int32, 16)

            @pl.loop(0, _GW, step=16)
            def _(c):
                lane = jnp.bitwise_and(i_vmem[0, pl.ds(c, 16)], 15)
                ofst_vmem[pl.ds(c, 16)] = plsc.load_gather(
                    rows_v, [lane16 + c, lane])

        pltpu.emit_pipeline(
            body,
            grid=(n // _GW,),
            in_specs=[pl.BlockSpec((1, _GW), lambda i: (0, i))],
            out_specs=[
                pl.BlockSpec((_GW, e), lambda i: (i, 0)),
                pl.BlockSpec((_GW,), lambda i: (i,)),
            ],
            core_axis_name=("c", "s"),
            dimension_semantics=(pltpu.PARALLEL,),
        )(i_hbm, osec_hbm, ofst_hbm)

    return k(sec_flat, fst16, idx)


def _tc_block(f, e, secg_ref, fstg_ref, xv_ref, w1_ref, b1_ref, w2_ref,
              b2_ref, bias_ref, out_ref):
    hi = lax.Precision.HIGHEST
    secg = secg_ref[...]
    xv = xv_ref[...]

    fi = lax.broadcasted_iota(jnp.int32, (f, f * e), 0)
    li = lax.broadcasted_iota(jnp.int32, (f, f * e), 1)
    erep = (li // e == fi).astype(jnp.float32)
    sec = secg * jnp.dot(xv, erep, precision=hi)

    l2 = lax.broadcasted_iota(jnp.int32, (f * e, e), 0)
    e2 = lax.broadcasted_iota(jnp.int32, (f * e, e), 1)
    srep = (l2 % e == e2).astype(jnp.float32)
    sum_f = jnp.dot(sec, srep, precision=hi)
    sum_sq = jnp.dot(sec * sec, srep, precision=hi)
    fm = 0.5 * (sum_f * sum_f - sum_sq)

    h = jnp.maximum(jnp.dot(sec, w1_ref[...], precision=hi) + b1_ref[...], 0.0)
    d = jnp.maximum(jnp.dot(h, w2_ref[...], precision=hi) + b2_ref[...], 0.0)

    fst_sum = jnp.sum(fstg_ref[...] * xv, axis=1, keepdims=True)
    out_ref[...] = (fst_sum + jnp.sum(fm, axis=1, keepdims=True)
                    + jnp.sum(d, axis=1, keepdims=True) + bias_ref[...])


def kernel(Xi, Xv, fst_tables, sec_tables, W1, b1, W2, b2, bias):
    b_sz, f, _ = Xi.shape
    v = sec_tables.shape[1]
    e = sec_tables.shape[2]
    h1 = W1.shape[1]
    h2 = W2.shape[1]

    idx = (Xi[:, :, 0].astype(jnp.int32)
           + jnp.arange(f, dtype=jnp.int32)[None, :] * v).reshape(1, b_sz * f)
    sec_flat = sec_tables.reshape(f * v, e)
    fst16 = fst_tables.reshape(f * v // 16, 16)

    sec_g, fst_g = _sc_gather(sec_flat, fst16, idx)
    sec_g = sec_g.reshape(b_sz, f * e)
    fst_g = fst_g.reshape(b_sz, f)
    xv = Xv[:, :, 0]

    bt = 1024
    out = pl.pallas_call(
        functools.partial(_tc_block, f, e),
        grid=(b_sz // bt,),
        in_specs=[
            pl.BlockSpec((bt, f * e), lambda i: (i, 0)),
            pl.BlockSpec((bt, f), lambda i: (i, 0)),
            pl.BlockSpec((bt, f), lambda i: (i, 0)),
            pl.BlockSpec((f * e, h1), lambda i: (0, 0)),
            pl.BlockSpec((1, h1), lambda i: (0, 0)),
            pl.BlockSpec((h1, h2), lambda i: (0, 0)),
            pl.BlockSpec((1, h2), lambda i: (0, 0)),
            pl.BlockSpec((1, 1), lambda i: (0, 0)),
        ],
        out_specs=pl.BlockSpec((bt, 1), lambda i: (i, 0)),
        out_shape=jax.ShapeDtypeStruct((b_sz, 1), jnp.float32),
    )(sec_g, fst_g, xv, W1, b1.reshape(1, h1), W2, b2.reshape(1, h2),
      bias.reshape(1, 1))
    return out.reshape(b_sz)
